# baseline (speedup 1.0000x reference)
import functools

import jax
import jax.numpy as jnp
from jax import lax
from jax.experimental import pallas as pl
from jax.experimental.pallas import tpu as pltpu
from jax.experimental.pallas import tpu_sc as plsc

_NC = 2
_NS = 16
_NW = _NC * _NS
_CHUNK = 128


def _sc_gather_body(nchunks, d,
                    uidx, gidx, iidx, utab, gtab, itab,
                    uout, gout, iout,
                    uidx_v, gidx_v, iidx_v, urows, grows, irows, sem):
    wid = lax.axis_index("s") * _NC + lax.axis_index("c")
    row_base = wid * (nchunks * _CHUNK)
    chunk_base = wid * nchunks
    pltpu.sync_copy(uidx.at[pl.ds(chunk_base, nchunks)], uidx_v)
    pltpu.sync_copy(gidx.at[pl.ds(chunk_base, nchunks)], gidx_v)
    pltpu.sync_copy(iidx.at[pl.ds(chunk_base, nchunks)], iidx_v)
    copies = []
    for j in range(nchunks):
        dst = pl.ds(j * _CHUNK, _CHUNK)
        copies.append(pltpu.async_copy(utab.at[uidx_v.at[j]], urows.at[dst], sem))
        copies.append(pltpu.async_copy(gtab.at[gidx_v.at[j]], grows.at[dst], sem))
        copies.append(pltpu.async_copy(itab.at[iidx_v.at[j]], irows.at[dst], sem))
    for c in copies:
        c.wait()
    out_slice = pl.ds(row_base, nchunks * _CHUNK)
    pltpu.sync_copy(urows, uout.at[out_slice])
    pltpu.sync_copy(grows, gout.at[out_slice])
    pltpu.sync_copy(irows, iout.at[out_slice])


def _sc_gather(uidx, gidx, iidx, utab, gtab, itab):
    b = uidx.shape[0]
    d = utab.shape[1]
    assert b % (_NW * _CHUNK) == 0
    nchunks = b // (_NW * _CHUNK)
    bw = nchunks * _CHUNK
    mesh = plsc.VectorSubcoreMesh(core_axis_name="c", subcore_axis_name="s")
    out_sds = jax.ShapeDtypeStruct((b, d), jnp.float32)
    fn = pl.kernel(
        functools.partial(_sc_gather_body, nchunks, d),
        out_type=(out_sds, out_sds, out_sds),
        mesh=mesh,
        scratch_types=[
            pltpu.VMEM((nchunks, _CHUNK), jnp.int32),
            pltpu.VMEM((nchunks, _CHUNK), jnp.int32),
            pltpu.VMEM((nchunks, _CHUNK), jnp.int32),
            pltpu.VMEM((bw, d), jnp.float32),
            pltpu.VMEM((bw, d), jnp.float32),
            pltpu.VMEM((bw, d), jnp.float32),
            pltpu.SemaphoreType.DMA,
        ],
        compiler_params=pltpu.CompilerParams(use_tc_tiling_on_sc=False),
    )
    u2 = uidx.reshape(b // _CHUNK, _CHUNK)
    g2 = gidx.reshape(b // _CHUNK, _CHUNK)
    i2 = iidx.reshape(b // _CHUNK, _CHUNK)
    return fn(u2, g2, i2, utab, gtab, itab)


def _mlp_body(u_ref, g_ref, i_ref, w1u_ref, w1g_ref, w1i_ref, b1_ref,
              w2_ref, b2_ref, w3_ref, b3_ref, out_ref):
    h = (u_ref[...] @ w1u_ref[...]
         + g_ref[...] @ w1g_ref[...]
         + i_ref[...] @ w1i_ref[...]
         + b1_ref[...])
    h = jnp.maximum(h, 0.0)
    h2 = jnp.maximum(h @ w2_ref[...] + b2_ref[...], 0.0)
    o = h2 @ w3_ref[...] + b3_ref[...]
    out_ref[...] = 1.0 / (1.0 + jnp.exp(-o))


def _mlp(u, g, i, W1, b1, W2, b2, W3, b3, blk=2048):
    b = u.shape[0]
    d = u.shape[1]
    grid = b // blk
    w1u = W1[0 * d:1 * d]
    w1g = W1[1 * d:2 * d]
    w1i = W1[2 * d:3 * d]
    row_spec = pl.BlockSpec((blk, d), lambda ib: (ib, 0))

    def full(a):
        return pl.BlockSpec(a.shape, lambda ib: (0,) * a.ndim)

    return pl.pallas_call(
        _mlp_body,
        grid=(grid,),
        in_specs=[row_spec, row_spec, row_spec,
                  full(w1u), full(w1g), full(w1i), full(b1),
                  full(W2), full(b2), full(W3), full(b3)],
        out_specs=pl.BlockSpec((blk, 1), lambda ib: (ib, 0)),
        out_shape=jax.ShapeDtypeStruct((b, 1), jnp.float32),
    )(u, g, i, w1u, w1g, w1i, b1, W2, b2, W3, b3)


def kernel(user_input, genre_input, item_input, user_table, genre_table,
           item_table, W1, b1, W2, b2, W3, b3):
    u, g, i = _sc_gather(user_input, genre_input, item_input,
                         user_table, genre_table, item_table)
    return _mlp(u, g, i, W1, b1, W2, b2, W3, b3)
